# Initial kernel scaffold; baseline (speedup 1.0000x reference)
import jax
import jax.numpy as jnp
from jax.experimental import pallas as pl


def kernel(text, image, table, W, b):
    raise NotImplementedError("write your pallas kernel here")


# speedup vs baseline: 2.7933x; 2.7933x over previous
import functools

import jax
import jax.numpy as jnp
from jax import lax
from jax.experimental import pallas as pl
from jax.experimental.pallas import tpu as pltpu
from jax.experimental.pallas import tpu_sc as plsc

BATCH = 4096
HIST = 50
DIM = 128
NC = 2
NS = 16
NW = NC * NS
N_IDX = BATCH * HIST
PER_W = N_IDX // NW
CHUNK = 128
NCH = PER_W // CHUNK


def _make_gather():
    mesh = plsc.VectorSubcoreMesh(core_axis_name="c", subcore_axis_name="s")

    @functools.partial(
        pl.kernel,
        mesh=mesh,
        out_type=jax.ShapeDtypeStruct((N_IDX, DIM), jnp.float32),
        scratch_types=[
            pltpu.VMEM((NCH, CHUNK), jnp.int32),
            pltpu.VMEM((CHUNK, DIM), jnp.float32),
            pltpu.VMEM((CHUNK, DIM), jnp.float32),
            pltpu.SemaphoreType.DMA,
            pltpu.SemaphoreType.DMA,
        ],
    )
    def gather_k(idx_hbm, table_hbm, out_hbm, idx_v, buf0, buf1, gsem, osem):
        wid = lax.axis_index("s") * NC + lax.axis_index("c")
        base = wid * PER_W
        pltpu.sync_copy(idx_hbm.at[wid], idx_v)

        def step(c, carry):
            pltpu.async_copy(table_hbm.at[idx_v.at[c]], buf0, gsem).wait()
            pltpu.sync_copy(buf0, out_hbm.at[pl.ds(base + c * CHUNK, CHUNK)])
            return carry

        lax.fori_loop(0, NCH, step, 0, unroll=False)

    return gather_k


_gather = _make_gather()


def _linear_body(x_ref, w_ref, b_ref, o_ref):
    o_ref[...] = (
        jnp.dot(x_ref[...], w_ref[...], preferred_element_type=jnp.float32)
        + b_ref[...]
    )


def _linear(image, W, b2d):
    blk = 512
    return pl.pallas_call(
        _linear_body,
        grid=(BATCH // blk,),
        in_specs=[
            pl.BlockSpec((blk, DIM), lambda i: (i, 0)),
            pl.BlockSpec((DIM, DIM), lambda i: (0, 0)),
            pl.BlockSpec((1, DIM), lambda i: (0, 0)),
        ],
        out_specs=pl.BlockSpec((blk, DIM), lambda i: (i, 0)),
        out_shape=jax.ShapeDtypeStruct((BATCH, DIM), jnp.float32),
    )(image, W, b2d)


def kernel(text, image, table, W, b):
    idx = text.reshape(NW, NCH, CHUNK).astype(jnp.int32)
    rows = _gather(idx, table)
    text_embedding = rows.reshape(BATCH, HIST, DIM)
    image_embedding = _linear(image, W, b.reshape(1, DIM))
    return (text_embedding, image_embedding)


# speedup vs baseline: 3.4169x; 1.2232x over previous
import functools

import jax
import jax.numpy as jnp
from jax import lax
from jax.experimental import pallas as pl
from jax.experimental.pallas import tpu as pltpu
from jax.experimental.pallas import tpu_sc as plsc

BATCH = 4096
HIST = 50
DIM = 128
NC = 2
NS = 16
NW = NC * NS
N_IDX = BATCH * HIST
PER_W = N_IDX // NW
CHUNK = 128
NCH = PER_W // CHUNK


VOCAB = 1000


def _make_gather():
    mesh = plsc.VectorSubcoreMesh(core_axis_name="c", subcore_axis_name="s")

    @functools.partial(
        pl.kernel,
        mesh=mesh,
        out_type=jax.ShapeDtypeStruct((N_IDX, DIM), jnp.float32),
        scratch_types=[
            pltpu.VMEM((NCH, CHUNK), jnp.int32),
            pltpu.VMEM((CHUNK, DIM), jnp.float32),
            pltpu.VMEM((CHUNK, DIM), jnp.float32),
            pltpu.VMEM_SHARED((VOCAB, DIM), jnp.float32),
            pltpu.SemaphoreType.DMA,
            pltpu.SemaphoreType.DMA,
            pltpu.SemaphoreType.DMA,
            pltpu.SemaphoreType.DMA,
        ],
    )
    def gather_k(idx_hbm, table_hbm, out_hbm, idx_v, buf0, buf1, tab_s,
                 gsem0, gsem1, osem0, osem1):
        sid = lax.axis_index("s")
        wid = sid * NC + lax.axis_index("c")
        base = wid * PER_W

        @pl.when(sid == 0)
        def _():
            pltpu.sync_copy(table_hbm, tab_s)

        pltpu.sync_copy(idx_hbm.at[wid], idx_v)
        plsc.subcore_barrier()

        pltpu.async_copy(tab_s.at[idx_v.at[0]], buf0, gsem0)
        pltpu.async_copy(tab_s.at[idx_v.at[1]], buf1, gsem1)

        def step(i, carry):
            c0 = 2 * i
            pltpu.make_async_copy(tab_s.at[idx_v.at[c0]], buf0, gsem0).wait()
            o0 = pltpu.async_copy(
                buf0, out_hbm.at[pl.ds(base + c0 * CHUNK, CHUNK)], osem0)
            pltpu.make_async_copy(tab_s.at[idx_v.at[c0]], buf1, gsem1).wait()
            o1 = pltpu.async_copy(
                buf1, out_hbm.at[pl.ds(base + (c0 + 1) * CHUNK, CHUNK)], osem1)

            @pl.when(i < NCH // 2 - 1)
            def _():
                o0.wait()
                pltpu.async_copy(tab_s.at[idx_v.at[c0 + 2]], buf0, gsem0)
                o1.wait()
                pltpu.async_copy(tab_s.at[idx_v.at[c0 + 3]], buf1, gsem1)

            return carry

        lax.fori_loop(0, NCH // 2, step, 0, unroll=False)

        pltpu.make_async_copy(
            buf0, out_hbm.at[pl.ds(base + (NCH - 2) * CHUNK, CHUNK)],
            osem0).wait()
        pltpu.make_async_copy(
            buf1, out_hbm.at[pl.ds(base + (NCH - 1) * CHUNK, CHUNK)],
            osem1).wait()

    return gather_k


_gather = _make_gather()


def _linear_body(x_ref, w_ref, b_ref, o_ref):
    o_ref[...] = (
        jnp.dot(x_ref[...], w_ref[...], preferred_element_type=jnp.float32)
        + b_ref[...]
    )


def _linear(image, W, b2d):
    blk = 512
    return pl.pallas_call(
        _linear_body,
        grid=(BATCH // blk,),
        in_specs=[
            pl.BlockSpec((blk, DIM), lambda i: (i, 0)),
            pl.BlockSpec((DIM, DIM), lambda i: (0, 0)),
            pl.BlockSpec((1, DIM), lambda i: (0, 0)),
        ],
        out_specs=pl.BlockSpec((blk, DIM), lambda i: (i, 0)),
        out_shape=jax.ShapeDtypeStruct((BATCH, DIM), jnp.float32),
    )(image, W, b2d)


def kernel(text, image, table, W, b):
    idx = text.reshape(NW, NCH, CHUNK).astype(jnp.int32)
    rows = _gather(idx, table)
    text_embedding = rows.reshape(BATCH, HIST, DIM)
    image_embedding = _linear(image, W, b.reshape(1, DIM))
    return (text_embedding, image_embedding)


# speedup vs baseline: 6.1979x; 1.8139x over previous
import functools

import jax
import jax.numpy as jnp
from jax import lax
from jax.experimental import pallas as pl
from jax.experimental.pallas import tpu as pltpu
from jax.experimental.pallas import tpu_sc as plsc

BATCH = 4096
HIST = 50
DIM = 128
NC = 2
NS = 16
NW = NC * NS
N_IDX = BATCH * HIST
PER_W = N_IDX // NW
CHUNK = 128
NCH = PER_W // CHUNK


VOCAB = 1000
B_PER_W = BATCH // NW
BCHUNK = 2
CROWS = BCHUNK * HIST
NCH3 = B_PER_W // BCHUNK


def _make_gather():
    mesh = plsc.VectorSubcoreMesh(core_axis_name="c", subcore_axis_name="s")

    @functools.partial(
        pl.kernel,
        mesh=mesh,
        out_type=jax.ShapeDtypeStruct((BATCH, HIST, DIM), jnp.float32),
        scratch_types=[
            pltpu.VMEM((NCH3, CROWS), jnp.int32),
            pltpu.VMEM((CROWS, DIM), jnp.float32),
            pltpu.VMEM((CROWS, DIM), jnp.float32),
            pltpu.VMEM_SHARED((VOCAB, DIM), jnp.float32),
            pltpu.SemaphoreType.DMA,
            pltpu.SemaphoreType.DMA,
            pltpu.SemaphoreType.DMA,
            pltpu.SemaphoreType.DMA,
        ],
    )
    def gather_k(idx_hbm, table_hbm, out_hbm, idx_v, buf0, buf1, tab_s,
                 gsem0, gsem1, osem0, osem1):
        sid = lax.axis_index("s")
        wid = sid * NC + lax.axis_index("c")
        wb = wid * B_PER_W

        @pl.when(sid == 0)
        def _():
            pltpu.sync_copy(table_hbm, tab_s)

        pltpu.sync_copy(idx_hbm.at[wid], idx_v)
        plsc.subcore_barrier()

        def write_out(buf, j, osem):
            pltpu.async_copy(
                buf.at[pl.ds(0, HIST)], out_hbm.at[wb + BCHUNK * j], osem)
            pltpu.async_copy(
                buf.at[pl.ds(HIST, HIST)],
                out_hbm.at[wb + BCHUNK * j + 1], osem)

        def wait_out(buf, j, osem):
            pltpu.make_async_copy(
                buf.at[pl.ds(0, HIST)], out_hbm.at[wb + BCHUNK * j],
                osem).wait()
            pltpu.make_async_copy(
                buf.at[pl.ds(HIST, HIST)],
                out_hbm.at[wb + BCHUNK * j + 1], osem).wait()

        pltpu.async_copy(tab_s.at[idx_v.at[0]], buf0, gsem0)
        pltpu.async_copy(tab_s.at[idx_v.at[1]], buf1, gsem1)

        def step(i, carry):
            j0 = 2 * i
            pltpu.make_async_copy(tab_s.at[idx_v.at[j0]], buf0, gsem0).wait()
            write_out(buf0, j0, osem0)
            pltpu.make_async_copy(tab_s.at[idx_v.at[j0]], buf1, gsem1).wait()
            write_out(buf1, j0 + 1, osem1)

            @pl.when(i < NCH3 // 2 - 1)
            def _():
                wait_out(buf0, j0, osem0)
                pltpu.async_copy(tab_s.at[idx_v.at[j0 + 2]], buf0, gsem0)
                wait_out(buf1, j0 + 1, osem1)
                pltpu.async_copy(tab_s.at[idx_v.at[j0 + 3]], buf1, gsem1)

            return carry

        lax.fori_loop(0, NCH3 // 2, step, 0, unroll=False)

        wait_out(buf0, NCH3 - 2, osem0)
        wait_out(buf1, NCH3 - 1, osem1)

    return gather_k


_gather = _make_gather()


def _linear_body(x_ref, w_ref, b_ref, o_ref):
    o_ref[...] = (
        jnp.dot(x_ref[...], w_ref[...], preferred_element_type=jnp.float32)
        + b_ref[...]
    )


def _linear(image, W, b2d):
    blk = 512
    return pl.pallas_call(
        _linear_body,
        grid=(BATCH // blk,),
        in_specs=[
            pl.BlockSpec((blk, DIM), lambda i: (i, 0)),
            pl.BlockSpec((DIM, DIM), lambda i: (0, 0)),
            pl.BlockSpec((1, DIM), lambda i: (0, 0)),
        ],
        out_specs=pl.BlockSpec((blk, DIM), lambda i: (i, 0)),
        out_shape=jax.ShapeDtypeStruct((BATCH, DIM), jnp.float32),
    )(image, W, b2d)


def kernel(text, image, table, W, b):
    idx = text.reshape(NW, NCH3, CROWS).astype(jnp.int32)
    text_embedding = _gather(idx, table)
    image_embedding = _linear(image, W, b.reshape(1, DIM))
    return (text_embedding, image_embedding)


# speedup vs baseline: 6.1984x; 1.0001x over previous
import functools

import jax
import jax.numpy as jnp
from jax import lax
from jax.experimental import pallas as pl
from jax.experimental.pallas import tpu as pltpu
from jax.experimental.pallas import tpu_sc as plsc

BATCH = 4096
HIST = 50
DIM = 128
NC = 2
NS = 16
NW = NC * NS
N_IDX = BATCH * HIST
PER_W = N_IDX // NW
CHUNK = 128
NCH = PER_W // CHUNK


VOCAB = 1000
B_PER_W = BATCH // NW
BCHUNK = 2
CROWS = BCHUNK * HIST
NCH3 = B_PER_W // BCHUNK


def _make_gather():
    mesh = plsc.VectorSubcoreMesh(core_axis_name="c", subcore_axis_name="s")

    @functools.partial(
        pl.kernel,
        mesh=mesh,
        out_type=jax.ShapeDtypeStruct((BATCH, HIST, DIM), jnp.float32),
        compiler_params=pltpu.CompilerParams(use_tc_tiling_on_sc=True),
        scratch_types=[
            pltpu.VMEM((NCH3, CROWS), jnp.int32),
            pltpu.VMEM((CROWS, DIM), jnp.float32),
            pltpu.VMEM((CROWS, DIM), jnp.float32),
            pltpu.VMEM_SHARED((VOCAB, DIM), jnp.float32),
            pltpu.SemaphoreType.DMA,
            pltpu.SemaphoreType.DMA,
            pltpu.SemaphoreType.DMA,
            pltpu.SemaphoreType.DMA,
        ],
    )
    def gather_k(idx_hbm, table_hbm, out_hbm, idx_v, buf0, buf1, tab_s,
                 gsem0, gsem1, osem0, osem1):
        sid = lax.axis_index("s")
        wid = sid * NC + lax.axis_index("c")
        wb = wid * B_PER_W

        @pl.when(sid == 0)
        def _():
            pltpu.sync_copy(table_hbm, tab_s)

        pltpu.sync_copy(idx_hbm.at[wid], idx_v)
        plsc.subcore_barrier()

        def write_out(buf, j, osem):
            pltpu.async_copy(
                buf.at[pl.ds(0, HIST)], out_hbm.at[wb + BCHUNK * j], osem)
            pltpu.async_copy(
                buf.at[pl.ds(HIST, HIST)],
                out_hbm.at[wb + BCHUNK * j + 1], osem)

        def wait_out(buf, j, osem):
            pltpu.make_async_copy(
                buf.at[pl.ds(0, HIST)], out_hbm.at[wb + BCHUNK * j],
                osem).wait()
            pltpu.make_async_copy(
                buf.at[pl.ds(HIST, HIST)],
                out_hbm.at[wb + BCHUNK * j + 1], osem).wait()

        pltpu.async_copy(tab_s.at[idx_v.at[0]], buf0, gsem0)
        pltpu.async_copy(tab_s.at[idx_v.at[1]], buf1, gsem1)

        def step(i, carry):
            j0 = 2 * i
            pltpu.make_async_copy(tab_s.at[idx_v.at[j0]], buf0, gsem0).wait()
            write_out(buf0, j0, osem0)
            pltpu.make_async_copy(tab_s.at[idx_v.at[j0]], buf1, gsem1).wait()
            write_out(buf1, j0 + 1, osem1)

            @pl.when(i < NCH3 // 2 - 1)
            def _():
                wait_out(buf0, j0, osem0)
                pltpu.async_copy(tab_s.at[idx_v.at[j0 + 2]], buf0, gsem0)
                wait_out(buf1, j0 + 1, osem1)
                pltpu.async_copy(tab_s.at[idx_v.at[j0 + 3]], buf1, gsem1)

            return carry

        lax.fori_loop(0, NCH3 // 2, step, 0, unroll=False)

        wait_out(buf0, NCH3 - 2, osem0)
        wait_out(buf1, NCH3 - 1, osem1)

    return gather_k


_gather = _make_gather()


def _linear_body(x_ref, w_ref, b_ref, o_ref):
    o_ref[...] = (
        jnp.dot(x_ref[...], w_ref[...], preferred_element_type=jnp.float32)
        + b_ref[...]
    )


def _linear(image, W, b2d):
    blk = 512
    return pl.pallas_call(
        _linear_body,
        grid=(BATCH // blk,),
        in_specs=[
            pl.BlockSpec((blk, DIM), lambda i: (i, 0)),
            pl.BlockSpec((DIM, DIM), lambda i: (0, 0)),
            pl.BlockSpec((1, DIM), lambda i: (0, 0)),
        ],
        out_specs=pl.BlockSpec((blk, DIM), lambda i: (i, 0)),
        out_shape=jax.ShapeDtypeStruct((BATCH, DIM), jnp.float32),
    )(image, W, b2d)


def kernel(text, image, table, W, b):
    idx = text.reshape(NW, NCH3, CROWS).astype(jnp.int32)
    text_embedding = _gather(idx, table)
    image_embedding = _linear(image, W, b.reshape(1, DIM))
    return (text_embedding, image_embedding)


# speedup vs baseline: 11.2950x; 1.8222x over previous
import functools

import jax
import jax.numpy as jnp
from jax import lax
from jax.experimental import pallas as pl
from jax.experimental.pallas import tpu as pltpu
from jax.experimental.pallas import tpu_sc as plsc

BATCH = 4096
HIST = 50
DIM = 128
NC = 2
NS = 16
NW = NC * NS
N_IDX = BATCH * HIST
PER_W = N_IDX // NW
CHUNK = 128
NCH = PER_W // CHUNK


VOCAB = 1000


def _make_gather():
    mesh = plsc.VectorSubcoreMesh(core_axis_name="c", subcore_axis_name="s")

    @functools.partial(
        pl.kernel,
        mesh=mesh,
        out_type=jax.ShapeDtypeStruct((N_IDX, DIM), jnp.float32),
        scratch_types=[
            pltpu.VMEM((NCH, CHUNK), jnp.int32),
            pltpu.VMEM((CHUNK, DIM), jnp.float32),
            pltpu.VMEM((CHUNK, DIM), jnp.float32),
            pltpu.VMEM_SHARED((VOCAB, DIM), jnp.float32),
            pltpu.SemaphoreType.DMA,
            pltpu.SemaphoreType.DMA,
            pltpu.SemaphoreType.DMA,
            pltpu.SemaphoreType.DMA,
        ],
    )
    def gather_k(idx_hbm, table_hbm, out_hbm, idx_v, buf0, buf1, tab_s,
                 gsem0, gsem1, osem0, osem1):
        sid = lax.axis_index("s")
        wid = sid * NC + lax.axis_index("c")
        base = wid * PER_W

        @pl.when(sid == 0)
        def _():
            pltpu.sync_copy(table_hbm, tab_s)

        pltpu.sync_copy(idx_hbm.at[wid], idx_v)
        plsc.subcore_barrier()

        def out_slice(c):
            return out_hbm.at[pl.ds(base + c * CHUNK, CHUNK)]

        pltpu.async_copy(tab_s.at[idx_v.at[0]], buf0, gsem0)
        pltpu.async_copy(tab_s.at[idx_v.at[1]], buf1, gsem1)

        def step(i, carry):
            c0 = 2 * i
            pltpu.make_async_copy(tab_s.at[idx_v.at[c0]], buf0, gsem0).wait()
            pltpu.async_copy(buf0, out_slice(c0), osem0)
            pltpu.make_async_copy(tab_s.at[idx_v.at[c0]], buf1, gsem1).wait()
            pltpu.async_copy(buf1, out_slice(c0 + 1), osem1)

            @pl.when(i < NCH // 2 - 1)
            def _():
                pltpu.make_async_copy(buf0, out_slice(c0), osem0).wait()
                pltpu.async_copy(tab_s.at[idx_v.at[c0 + 2]], buf0, gsem0)
                pltpu.make_async_copy(buf1, out_slice(c0 + 1), osem1).wait()
                pltpu.async_copy(tab_s.at[idx_v.at[c0 + 3]], buf1, gsem1)

            return carry

        lax.fori_loop(0, NCH // 2, step, 0, unroll=False)

        pltpu.make_async_copy(buf0, out_slice(NCH - 2), osem0).wait()
        pltpu.make_async_copy(buf1, out_slice(NCH - 1), osem1).wait()

    return gather_k


_gather = _make_gather()


def _linear_body(x_ref, w_ref, b_ref, o_ref):
    o_ref[...] = (
        jnp.dot(x_ref[...], w_ref[...], preferred_element_type=jnp.float32)
        + b_ref[...]
    )


def _linear(image, W, b2d):
    blk = 512
    return pl.pallas_call(
        _linear_body,
        grid=(BATCH // blk,),
        in_specs=[
            pl.BlockSpec((blk, DIM), lambda i: (i, 0)),
            pl.BlockSpec((DIM, DIM), lambda i: (0, 0)),
            pl.BlockSpec((1, DIM), lambda i: (0, 0)),
        ],
        out_specs=pl.BlockSpec((blk, DIM), lambda i: (i, 0)),
        out_shape=jax.ShapeDtypeStruct((BATCH, DIM), jnp.float32),
    )(image, W, b2d)


def kernel(text, image, table, W, b):
    idx = text.astype(jnp.int32).T.reshape(NW, NCH, CHUNK)
    rows = _gather(idx, table)
    text_embedding = rows.reshape(HIST, BATCH, DIM).transpose(1, 0, 2)
    image_embedding = _linear(image, W, b.reshape(1, DIM))
    return (text_embedding, image_embedding)


# speedup vs baseline: 15.2067x; 1.3463x over previous
import functools

import jax
import jax.numpy as jnp
from jax import lax
from jax.experimental import pallas as pl
from jax.experimental.pallas import tpu as pltpu
from jax.experimental.pallas import tpu_sc as plsc

BATCH = 4096
HIST = 50
DIM = 128
NC = 2
NS = 16
NW = NC * NS
N_IDX = BATCH * HIST
PER_W = N_IDX // NW
CHUNK = 128
NCH = PER_W // CHUNK


VOCAB = 1000
NBUF = 5
STAGE_T = 8
STAGE_R = VOCAB // STAGE_T


def _make_gather():
    mesh = plsc.VectorSubcoreMesh(core_axis_name="c", subcore_axis_name="s")

    @functools.partial(
        pl.kernel,
        mesh=mesh,
        out_type=jax.ShapeDtypeStruct((N_IDX, DIM), jnp.float32),
        scratch_types=[
            pltpu.VMEM((NCH, CHUNK), jnp.int32),
        ] + [pltpu.VMEM((CHUNK, DIM), jnp.float32) for _ in range(NBUF)] + [
            pltpu.VMEM_SHARED((VOCAB, DIM), jnp.float32),
        ] + [pltpu.SemaphoreType.DMA for _ in range(2 * NBUF)],
    )
    def gather_k(idx_hbm, table_hbm, out_hbm, idx_v, *rest):
        bufs = rest[:NBUF]
        tab_s = rest[NBUF]
        gs = rest[NBUF + 1:2 * NBUF + 1]
        os_ = rest[2 * NBUF + 1:]
        sid = lax.axis_index("s")
        wid = sid * NC + lax.axis_index("c")
        base = wid * PER_W

        for t in range(STAGE_T):
            nrows = min(128, VOCAB - t * 128)

            @pl.when(sid == t)
            def _(t=t, nrows=nrows):
                pltpu.sync_copy(table_hbm.at[pl.ds(t * 128, nrows)],
                                tab_s.at[pl.ds(t * 128, nrows)])

        pltpu.sync_copy(idx_hbm.at[wid], idx_v)
        plsc.subcore_barrier()

        def out_slice(c):
            return out_hbm.at[pl.ds(base + c * CHUNK, CHUNK)]

        for bx in range(NBUF):
            pltpu.async_copy(tab_s.at[idx_v.at[bx]], bufs[bx], gs[bx])

        def step(i, carry):
            c0 = NBUF * i
            for bx in range(NBUF):
                pltpu.make_async_copy(
                    tab_s.at[idx_v.at[c0]], bufs[bx], gs[bx]).wait()
                pltpu.async_copy(bufs[bx], out_slice(c0 + bx), os_[bx])

            @pl.when(i < NCH // NBUF - 1)
            def _():
                for bx in range(NBUF):
                    pltpu.make_async_copy(
                        bufs[bx], out_slice(c0 + bx), os_[bx]).wait()
                    pltpu.async_copy(
                        tab_s.at[idx_v.at[c0 + NBUF + bx]], bufs[bx], gs[bx])

            return carry

        lax.fori_loop(0, NCH // NBUF, step, 0, unroll=False)

        for bx in range(NBUF):
            pltpu.make_async_copy(
                bufs[bx], out_slice(NCH - NBUF + bx), os_[bx]).wait()

    return gather_k


_gather = _make_gather()


def _linear_body(x_ref, w_ref, b_ref, o_ref):
    o_ref[...] = (
        jnp.dot(x_ref[...], w_ref[...], preferred_element_type=jnp.float32)
        + b_ref[...]
    )


def _linear(image, W, b2d):
    blk = 512
    return pl.pallas_call(
        _linear_body,
        grid=(BATCH // blk,),
        in_specs=[
            pl.BlockSpec((blk, DIM), lambda i: (i, 0)),
            pl.BlockSpec((DIM, DIM), lambda i: (0, 0)),
            pl.BlockSpec((1, DIM), lambda i: (0, 0)),
        ],
        out_specs=pl.BlockSpec((blk, DIM), lambda i: (i, 0)),
        out_shape=jax.ShapeDtypeStruct((BATCH, DIM), jnp.float32),
    )(image, W, b2d)


def kernel(text, image, table, W, b):
    idx = text.astype(jnp.int32).T.reshape(NW, NCH, CHUNK)
    rows = _gather(idx, table)
    text_embedding = rows.reshape(HIST, BATCH, DIM).transpose(1, 0, 2)
    image_embedding = _linear(image, W, b.reshape(1, DIM))
    return (text_embedding, image_embedding)
